# baseline (speedup 1.0000x reference)
import functools

import jax
import jax.numpy as jnp
from jax import lax
from jax.experimental import pallas as pl
from jax.experimental.pallas import tpu as pltpu
from jax.experimental.pallas import tpu_sc as plsc

NC = 2
NS = 16
NW = NC * NS
EC = 128
DW = 16
ZC = 208
ZB = 48


def _rows_out(acc, out_hbm, c, s, n, copy, chunk):
    rb = (n // NS) & ~7
    ex = n - NS * rb
    assert rb % chunk == 0 and ex % 8 == 0 and ex < chunk
    base = pl.multiple_of(s * rb, 8)
    for k in range(rb // chunk):
        copy(base + k * chunk, chunk, False)
    if ex:
        @pl.when(s == NS - 1)
        def _():
            copy(n - ex, ex, True)


NB = 3


def _deg_sc(dst, n):
    e = dst.shape[0]
    ept = e // NW
    nfull, tail = ept // EC, ept % EC
    assert e % NW == 0 and tail % 8 == 0 and ept % 8 == 0

    mesh = plsc.VectorSubcoreMesh(core_axis_name="c", subcore_axis_name="s")

    def body(dst_hbm, out_hbm, acc, ones_v, ones_t,
             didx0, didx_t, zbuf):
        c = lax.axis_index("c")
        s = lax.axis_index("s")
        wid = s * NC + c

        zero16 = jnp.zeros((16,), jnp.float32)
        one16 = jnp.ones((16,), jnp.float32)

        def fill(i, _):
            zbuf[i, :] = zero16
            return 0

        lax.fori_loop(0, ZB, fill, 0)

        def fill2(i, _):
            ones_v[i, :] = one16
            return 0

        lax.fori_loop(0, EC, fill2, 0)
        if tail:
            def fill3(i, _):
                ones_t[i, :] = one16
                return 0

            lax.fori_loop(0, tail, fill3, 0)

        def zcopy(off, cnt, is_tail):
            zsrc = zbuf.at[pl.ds(0, cnt)] if is_tail else zbuf
            pltpu.sync_copy(zsrc, acc.at[pl.ds(off, cnt)])

        _rows_out(acc, out_hbm, c, s, n, zcopy, ZB)
        plsc.subcore_barrier()

        eb = wid * ept

        def chunk(j, _):
            pltpu.sync_copy(dst_hbm.at[pl.ds(eb + j * EC, EC)], didx0)
            pltpu.sync_copy(ones_v, acc.at[didx0], add=True)
            return 0

        lax.fori_loop(0, nfull, chunk, 0)
        if tail:
            pltpu.sync_copy(dst_hbm.at[pl.ds(eb + nfull * EC, tail)], didx_t)
            pltpu.sync_copy(ones_t, acc.at[didx_t], add=True)

        plsc.subcore_barrier()

        def wcopy(off, cnt, is_tail):
            pltpu.sync_copy(acc.at[pl.ds(off, cnt)],
                            out_hbm.at[c, pl.ds(off, cnt)])

        _rows_out(acc, out_hbm, c, s, n, wcopy, ZC)

    scratch = [
        pltpu.VMEM_SHARED((n, DW), jnp.float32),
        pltpu.VMEM((EC, DW), jnp.float32),
        pltpu.VMEM((max(tail, 8), DW), jnp.float32),
        pltpu.VMEM((EC,), jnp.int32),
        pltpu.VMEM((max(tail, 8),), jnp.int32),
        pltpu.VMEM((ZB, DW), jnp.float32),
    ]
    return pl.kernel(
        body,
        jax.ShapeDtypeStruct((NC, n, DW), jnp.float32),
        mesh=mesh,
        scratch_types=scratch,
    )(dst)


def _agg_sc(hp, src, dst):
    n, d = hp.shape
    e = src.shape[0]
    ept = e // NW
    nfull, tail = ept // EC, ept % EC
    assert e % NW == 0 and d % 16 == 0
    assert tail % 8 == 0 and ept % 8 == 0
    assert nfull % 2 == 0

    mesh = plsc.VectorSubcoreMesh(core_axis_name="c", subcore_axis_name="s")

    def body(hp_hbm, src_hbm, dst_hbm, out_hbm, acc,
             rows0, rows1, sidx0, sidx1, didx0, didx1,
             rows_t, sidx_t, didx_t, zbuf,
             gsem0, gsem1, isem0, isem1, tsem):
        rows = [rows0, rows1]
        sidx = [sidx0, sidx1]
        didx = [didx0, didx1]
        gsem = [gsem0, gsem1]
        isem = [isem0, isem1]
        c = lax.axis_index("c")
        s = lax.axis_index("s")
        wid = s * NC + c

        zero16 = jnp.zeros((16,), jnp.float32)

        def fill(i, _):
            for q in range(d // 16):
                zbuf[i, pl.ds(q * 16, 16)] = zero16
            return 0

        lax.fori_loop(0, ZB, fill, 0)

        def zcopy(off, cnt, is_tail):
            zsrc = zbuf.at[pl.ds(0, cnt)] if is_tail else zbuf
            pltpu.sync_copy(zsrc, acc.at[pl.ds(off, cnt)])

        _rows_out(acc, out_hbm, c, s, n, zcopy, ZB)
        plsc.subcore_barrier()

        eb = wid * ept

        def pair(k, _):
            j0 = k * 2
            b0 = eb + j0 * EC
            b1 = eb + (j0 + 1) * EC
            pltpu.sync_copy(src_hbm.at[pl.ds(b0, EC)], sidx0)
            pltpu.sync_copy(dst_hbm.at[pl.ds(b0, EC)], didx0)
            pltpu.sync_copy(src_hbm.at[pl.ds(b1, EC)], sidx1)
            pltpu.sync_copy(dst_hbm.at[pl.ds(b1, EC)], didx1)
            da = pltpu.async_copy(hp_hbm.at[sidx0], rows0, gsem0)
            db = pltpu.async_copy(hp_hbm.at[sidx1], rows1, gsem1)
            da.wait()
            pltpu.sync_copy(rows0, acc.at[didx0], add=True)
            db.wait()
            pltpu.sync_copy(rows1, acc.at[didx1], add=True)
            return 0

        lax.fori_loop(0, nfull // 2, pair, 0)
        if tail:
            b0 = eb + nfull * EC
            pltpu.sync_copy(src_hbm.at[pl.ds(b0, tail)], sidx_t)
            pltpu.sync_copy(dst_hbm.at[pl.ds(b0, tail)], didx_t)
            pltpu.async_copy(hp_hbm.at[sidx_t], rows_t, tsem).wait()
            pltpu.sync_copy(rows_t, acc.at[didx_t], add=True)

        plsc.subcore_barrier()

        def wcopy(off, cnt, is_tail):
            pltpu.sync_copy(acc.at[pl.ds(off, cnt)],
                            out_hbm.at[c, pl.ds(off, cnt)])

        _rows_out(acc, out_hbm, c, s, n, wcopy, ZC)

    scratch = [
        pltpu.VMEM_SHARED((n, d), jnp.float32),
        pltpu.VMEM((EC, d), jnp.float32),
        pltpu.VMEM((EC, d), jnp.float32),
        pltpu.VMEM((EC,), jnp.int32),
        pltpu.VMEM((EC,), jnp.int32),
        pltpu.VMEM((EC,), jnp.int32),
        pltpu.VMEM((EC,), jnp.int32),
        pltpu.VMEM((max(tail, 8), d), jnp.float32),
        pltpu.VMEM((max(tail, 8),), jnp.int32),
        pltpu.VMEM((max(tail, 8),), jnp.int32),
        pltpu.VMEM((ZB, d), jnp.float32),
        pltpu.SemaphoreType.DMA,
        pltpu.SemaphoreType.DMA,
        pltpu.SemaphoreType.DMA,
        pltpu.SemaphoreType.DMA,
        pltpu.SemaphoreType.DMA,
    ]
    return pl.kernel(
        body,
        jax.ShapeDtypeStruct((NC, n, d), jnp.float32),
        mesh=mesh,
        scratch_types=scratch,
    )(hp, src, dst)


_TC_R = 1000


def _dinv_of(g):
    deg = g[0, :, :1] + g[1, :, :1] + 1.0
    return lax.rsqrt(deg)


def _tc_first(degp, x, w):
    n, d = x.shape

    def body(g_ref, x_ref, w_ref, o_ref):
        dinv = _dinv_of(g_ref[...])
        o_ref[...] = dinv * jnp.dot(x_ref[...], w_ref[...],
                                    preferred_element_type=jnp.float32)

    return pl.pallas_call(
        body,
        grid=(n // _TC_R,),
        in_specs=[
            pl.BlockSpec((2, _TC_R, DW), lambda i: (0, i, 0)),
            pl.BlockSpec((_TC_R, d), lambda i: (i, 0)),
            pl.BlockSpec((d, w.shape[1]), lambda i: (0, 0)),
        ],
        out_specs=pl.BlockSpec((_TC_R, w.shape[1]), lambda i: (i, 0)),
        out_shape=jax.ShapeDtypeStruct((n, w.shape[1]), jnp.float32),
    )(degp, x, w)


def _tc_mid(degp, tmpp, hp, b, w):
    n, d = hp.shape

    def body(g_ref, t_ref, hp_ref, b_ref, w_ref, o_ref):
        dinv = _dinv_of(g_ref[...])
        t = t_ref[...]
        h = jnp.maximum(dinv * (t[0] + t[1] + hp_ref[...]) + b_ref[...], 0.0)
        o_ref[...] = dinv * jnp.dot(h, w_ref[...],
                                    preferred_element_type=jnp.float32)

    return pl.pallas_call(
        body,
        grid=(n // _TC_R,),
        in_specs=[
            pl.BlockSpec((2, _TC_R, DW), lambda i: (0, i, 0)),
            pl.BlockSpec((2, _TC_R, d), lambda i: (0, i, 0)),
            pl.BlockSpec((_TC_R, d), lambda i: (i, 0)),
            pl.BlockSpec((1, d), lambda i: (0, 0)),
            pl.BlockSpec((d, w.shape[1]), lambda i: (0, 0)),
        ],
        out_specs=pl.BlockSpec((_TC_R, w.shape[1]), lambda i: (i, 0)),
        out_shape=jax.ShapeDtypeStruct((n, w.shape[1]), jnp.float32),
    )(degp, tmpp, hp, b, w)


def _tc_last(degp, tmpp, hp, b, wc, bc):
    n, d = hp.shape
    dout = wc.shape[1]

    def body(g_ref, t_ref, hp_ref, b_ref, w_ref, bc_ref, o_ref):
        dinv = _dinv_of(g_ref[...])
        t = t_ref[...]
        h = jnp.maximum(dinv * (t[0] + t[1] + hp_ref[...]) + b_ref[...], 0.0)
        o_ref[...] = jnp.dot(h, w_ref[...],
                             preferred_element_type=jnp.float32) + bc_ref[...]

    return pl.pallas_call(
        body,
        grid=(n // _TC_R,),
        in_specs=[
            pl.BlockSpec((2, _TC_R, DW), lambda i: (0, i, 0)),
            pl.BlockSpec((2, _TC_R, d), lambda i: (0, i, 0)),
            pl.BlockSpec((_TC_R, d), lambda i: (i, 0)),
            pl.BlockSpec((1, d), lambda i: (0, 0)),
            pl.BlockSpec((d, dout), lambda i: (0, 0)),
            pl.BlockSpec((1, dout), lambda i: (0, 0)),
        ],
        out_specs=pl.BlockSpec((_TC_R, dout), lambda i: (i, 0)),
        out_shape=jax.ShapeDtypeStruct((n, dout), jnp.float32),
    )(degp, tmpp, hp, b, wc, bc)


def kernel(x, edge_index, W1, b1, W2, b2, W3, b3, Wc, bc):
    src = edge_index[0]
    dst = edge_index[1]
    n = x.shape[0]

    degp = _deg_sc(dst, n)
    hp = _tc_first(degp, x, W1)
    for (b, wn) in ((b1, W2), (b2, W3)):
        tmpp = _agg_sc(hp, src, dst)
        hp = _tc_mid(degp, tmpp, hp, b.reshape(1, -1), wn)
    tmpp = _agg_sc(hp, src, dst)
    return _tc_last(degp, tmpp, hp, b3.reshape(1, -1), Wc, bc.reshape(1, -1))
